# baseline (speedup 1.0000x reference)
import functools

import numpy as np
import jax
import jax.numpy as jnp
from jax import lax
from jax.experimental import pallas as pl
from jax.experimental.pallas import tpu as pltpu
from jax.experimental.pallas import tpu_sc as plsc

_N = 10000
_NP = 10240
_ED = 64
_K = 16
_RB = 200
_NBLK = _N // _RB
_NW = 32
_PW = _NP // 16
_NG = _PW // 16


_L = 79
_C = 128
_NPAD = _L * _C
_DEPTH = 4


def _topk_body(eb_ref, ef_ref, out_ref):
    eb = eb_ref[...]
    ef = ef_ref[...]
    raw = lax.dot_general(eb, ef, (((1,), (1,)), ((), ())),
                          preferred_element_type=jnp.float32)
    nb = jnp.sqrt(jnp.sum(eb * eb, axis=1))
    nf = jnp.sqrt(jnp.sum(ef * ef, axis=1))
    nf = jnp.where(nf == 0.0, 1.0, nf)
    iotc = lax.broadcasted_iota(jnp.int32, (_RB, _NPAD), 1)
    cos = raw / (nb[:, None] * nf[None, :])
    cos = jnp.where(iotc >= _N, -2.0, cos)
    neg = jnp.float32(-jnp.inf)
    Ms, As = [], []
    for j in range(_DEPTH + 1):
        M = jnp.full((_RB, _C), neg, jnp.float32)
        A = jnp.zeros((_RB, _C), jnp.int32)
        for p in range(_L):
            s = cos[:, p * _C:(p + 1) * _C]
            valid = s > M
            for Aprev in As:
                valid = valid & (Aprev != p)
            M = jnp.where(valid, s, M)
            A = jnp.where(valid, p, A)
        Ms.append(M)
        As.append(A)
    M5 = Ms[_DEPTH]

    V, Ac = Ms[0], As[0]
    cnt = jnp.zeros((_RB, _C), jnp.int32)
    iota_c = lax.broadcasted_iota(jnp.int32, (_RB, _C), 1)
    cols = []
    vlast = None
    for _ in range(_K):
        cstar = jnp.argmax(V, axis=1).astype(jnp.int32)
        astar = jnp.take_along_axis(Ac, cstar[:, None], axis=1)
        vlast = jnp.take_along_axis(V, cstar[:, None], axis=1)
        cols.append(astar * _C + cstar[:, None])
        onehot = iota_c == cstar[:, None]
        cnt = cnt + onehot.astype(jnp.int32)
        nxtV = jnp.where(cnt == 1, Ms[1],
                         jnp.where(cnt == 2, Ms[2],
                                   jnp.where(cnt == 3, Ms[3], neg)))
        nxtA = jnp.where(cnt == 1, As[1],
                         jnp.where(cnt == 2, As[2], As[3]))
        V = jnp.where(onehot, nxtV, V)
        Ac = jnp.where(onehot, nxtA, Ac)
    out_ref[...] = jnp.concatenate(cols, axis=1)

    overflow = jnp.any((cnt >= _DEPTH) & (M5 > vlast))

    @pl.when(overflow)
    def _fallback():
        raw2 = lax.dot_general(eb, ef, (((1,), (1,)), ((), ())),
                               preferred_element_type=jnp.float32)
        cos2 = raw2 / (nb[:, None] * nf[None, :])
        cos2 = jnp.where(iotc >= _N, -2.0, cos2)
        colsf = []
        for _ in range(_K):
            idx = jnp.argmax(cos2, axis=1).astype(jnp.int32)
            colsf.append(idx[:, None])
            cos2 = jnp.where(iotc == idx[:, None], neg, cos2)
        out_ref[...] = jnp.concatenate(colsf, axis=1)


def _topk_stage(emb):
    emb_pad = jnp.pad(emb, ((0, _NPAD - _N), (0, 0)))
    return pl.pallas_call(
        _topk_body,
        grid=(_NBLK,),
        in_specs=[pl.BlockSpec((_RB, _ED), lambda i: (i, 0)),
                  pl.BlockSpec((_NPAD, _ED), lambda i: (0, 0))],
        out_specs=pl.BlockSpec((_RB, _K), lambda i: (i, 0)),
        out_shape=jax.ShapeDtypeStruct((_N, _K), jnp.int32),
    )(emb, emb_pad)


def _feat_body(x_ref, emb_ref, wg_ref, atts_ref, z_ref, ai_ref, aj_ref):
    x = x_ref[0]
    e = emb_ref[...]
    z = jnp.dot(x, wg_ref[...], preferred_element_type=jnp.float32)
    atts = atts_ref[...]
    ai = jnp.sum(z * atts[0][None, :], axis=1) + jnp.sum(e * atts[1][None, :], axis=1)
    aj = jnp.sum(z * atts[2][None, :], axis=1) + jnp.sum(e * atts[3][None, :], axis=1)
    z_ref[0] = z
    ai_ref[0, 0] = ai
    aj_ref[0, 0] = aj


def _feat_stage(data, emb, W_gat, atts):
    B_, N_, Ft = data.shape
    return pl.pallas_call(
        _feat_body,
        grid=(B_,),
        in_specs=[pl.BlockSpec((1, N_, Ft), lambda b: (b, 0, 0)),
                  pl.BlockSpec((N_, _ED), lambda b: (0, 0)),
                  pl.BlockSpec((Ft, _ED), lambda b: (0, 0)),
                  pl.BlockSpec((4, _ED), lambda b: (0, 0))],
        out_specs=[pl.BlockSpec((1, N_, _ED), lambda b: (b, 0, 0)),
                   pl.BlockSpec((1, 1, N_), lambda b: (b, 0, 0)),
                   pl.BlockSpec((1, 1, N_), lambda b: (b, 0, 0))],
        out_shape=[jax.ShapeDtypeStruct((B_, N_, _ED), jnp.float32),
                   jax.ShapeDtypeStruct((B_, 1, N_), jnp.float32),
                   jax.ShapeDtypeStruct((B_, 1, N_), jnp.float32)],
    )(data, emb, W_gat, atts)


def _gat_body(J_hbm, z_hbm, ai_hbm, aj_hbm, emb_hbm, epar_hbm, out_hbm,
              Jv, ajv, aiv, embv, zg, idxv, av, eparv, outv, sem):
    b = lax.axis_index("c")
    s = lax.axis_index("s")
    i0 = s * _PW
    pltpu.sync_copy(aj_hbm.at[b], ajv)
    pltpu.sync_copy(ai_hbm.at[b, pl.ds(i0, _PW)], aiv)
    pltpu.sync_copy(emb_hbm.at[pl.ds(i0, _PW)], embv)
    pltpu.sync_copy(epar_hbm, eparv)
    for k in range(_K):
        pltpu.sync_copy(J_hbm.at[k, pl.ds(i0, _PW)], Jv.at[k])
    iota16 = lax.iota(jnp.int32, 16)
    zoff = b * _NP

    def _start_gather(g_next, buf):
        for k in range(_K):
            jv = Jv[k, pl.ds(g_next * 16, 16)]
            idxv[buf, k // 8, pl.ds((k % 8) * 16, 16)] = jv + zoff
        for r in range(2):
            pltpu.async_copy(z_hbm.at[idxv.at[buf, r]], zg.at[buf, r], sem)

    def _wait_gather(buf):
        for r in range(2):
            pltpu.make_async_copy(
                z_hbm.at[idxv.at[buf, r]], zg.at[buf, r], sem).wait()

    _start_gather(0, 0)

    def _do_group(g, buf):
        gn = jnp.minimum(g + 1, _NG - 1)
        _start_gather(gn, 1 - buf)
        ai_v = aiv[pl.ds(g * 16, 16)]
        es = []
        for k in range(_K):
            jv = Jv[k, pl.ds(g * 16, 16)]
            ajg = plsc.load_gather(ajv, [jv])
            e = ai_v + ajg
            es.append(jnp.where(e > 0.0, e, 0.2 * e))
        m = es[0]
        for k in range(1, _K):
            m = jnp.maximum(m, es[k])
        exs = [jnp.exp(e - m) for e in es]
        ssum = exs[0]
        for k in range(1, _K):
            ssum = ssum + exs[k]
        rinv = 1.0 / ssum
        for k in range(_K):
            av[k] = exs[k] * rinv
        _wait_gather(buf)
        nloc = iota16 + g * 16

        def _epar(row, col):
            return plsc.load_gather(
                eparv, [jnp.full((16,), row, jnp.int32),
                        jnp.full((16,), col, jnp.int32)])

        bufc = jnp.full((16,), buf, jnp.int32)
        out_acc = _epar(4, 0)
        for cb in range(0, _ED, 4):
            accs = [jnp.zeros((16,), jnp.float32) for _ in range(4)]
            for k in range(_K):
                a = av[k]
                sub = jnp.full((16,), k // 8, jnp.int32)
                base = iota16 + (k % 8) * 16
                for j in range(4):
                    col = jnp.full((16,), cb + j, jnp.int32)
                    zc = plsc.load_gather(zg, [bufc, sub, base, col])
                    accs[j] = accs[j] + a * zc
            for j in range(4):
                c = cb + j
                col = jnp.full((16,), c, jnp.int32)
                gat_c = accs[j] + _epar(0, c)
                emb_c = plsc.load_gather(embv, [nloc, col])
                r = (gat_c * emb_c) * _epar(1, c) + _epar(2, c)
                r = jnp.maximum(r, 0.0)
                out_acc = out_acc + r * _epar(3, c)
        outv[pl.ds(g * 16, 16)] = out_acc

    def step(t, carry):
        _do_group(2 * t, 0)
        _do_group(2 * t + 1, 1)
        return carry

    lax.fori_loop(0, _NG // 2, step, 0)
    _wait_gather(0)
    pltpu.sync_copy(outv, out_hbm.at[b, pl.ds(i0, _PW)])


def _gat_call(*args):
    fn = functools.partial(
        pl.kernel,
        out_type=jax.ShapeDtypeStruct((2, _NP), jnp.float32),
        mesh=plsc.VectorSubcoreMesh(core_axis_name="c", subcore_axis_name="s"),
        compiler_params=pltpu.CompilerParams(needs_layout_passes=False,
                                             use_tc_tiling_on_sc=False),
        scratch_types=[
            pltpu.VMEM((_K, _PW), jnp.int32),
            pltpu.VMEM((_NP,), jnp.float32),
            pltpu.VMEM((_PW,), jnp.float32),
            pltpu.VMEM((_PW, _ED), jnp.float32),
            pltpu.VMEM((2, 2, 128, _ED), jnp.float32),
            pltpu.VMEM((2, 2, 128), jnp.int32),
            pltpu.VMEM((_K, 16), jnp.float32),
            pltpu.VMEM((8, _ED), jnp.float32),
            pltpu.VMEM((_PW,), jnp.float32),
            pltpu.SemaphoreType.DMA,
        ],
    )(_gat_body)
    return fn(*args)


def kernel(data, emb, W_gat, att_i, att_j, b_gat, gamma, beta, W_out, b_out):
    B_, N_, Ft = data.shape
    topk = _topk_stage(emb)
    atts = jnp.stack([att_i[:_ED], att_i[_ED:], att_j[:_ED], att_j[_ED:]])
    z, ai3, aj3 = _feat_stage(data, emb, W_gat, atts)
    ai2 = ai3.reshape(B_, N_)
    aj2 = aj3.reshape(B_, N_)
    pad = _NP - N_
    J = jnp.pad(topk.reshape(_K, N_), ((0, 0), (0, pad)))
    z_p = jnp.pad(z, ((0, 0), (0, pad), (0, 0))).reshape(B_ * _NP, _ED)
    ai_p = jnp.pad(ai2, ((0, 0), (0, pad)))
    aj_p = jnp.pad(aj2, ((0, 0), (0, pad)))
    emb_p = jnp.pad(emb, ((0, pad), (0, 0)))
    ginv = gamma * np.float32(1.0 / np.sqrt(1.0 + 1e-5))
    epar = jnp.stack([b_gat, ginv, beta, W_out[:, 0],
                      jnp.full((_ED,), b_out[0], jnp.float32),
                      jnp.zeros((_ED,), jnp.float32),
                      jnp.zeros((_ED,), jnp.float32),
                      jnp.zeros((_ED,), jnp.float32)])
    out_p = _gat_call(J, z_p, ai_p, aj_p, emb_p, epar)
    return out_p[:, :N_]


# speedup vs baseline: 1.2757x; 1.2757x over previous
import functools

import numpy as np
import jax
import jax.numpy as jnp
from jax import lax
from jax.experimental import pallas as pl
from jax.experimental.pallas import tpu as pltpu
from jax.experimental.pallas import tpu_sc as plsc

_N = 10000
_NP = 10240
_ED = 64
_K = 16
_RB = 200
_NBLK = _N // _RB
_NW = 32
_PW = _NP // 16
_NG = _PW // 16


_L = 79
_C = 128
_NPAD = _L * _C
_DEPTH = 4


def _topk_body(eb_ref, ef_ref, out_ref):
    eb = eb_ref[...]
    ef = ef_ref[...]
    raw = lax.dot_general(eb, ef, (((1,), (1,)), ((), ())),
                          preferred_element_type=jnp.float32)
    nb = jnp.sqrt(jnp.sum(eb * eb, axis=1))
    nf = jnp.sqrt(jnp.sum(ef * ef, axis=1))
    nf = jnp.where(nf == 0.0, 1.0, nf)
    iotc = lax.broadcasted_iota(jnp.int32, (_RB, _NPAD), 1)
    cos = raw / (nb[:, None] * nf[None, :])
    cos = jnp.where(iotc >= _N, -2.0, cos)
    work = cos.reshape(_RB, _L, _C)
    iota_l = lax.broadcasted_iota(jnp.int32, (_RB, _L, _C), 1)
    neg = jnp.float32(-jnp.inf)
    Ms, As = [], []
    for _ in range(_DEPTH):
        A = jnp.argmax(work, axis=1).astype(jnp.int32)
        M = jnp.max(work, axis=1)
        Ms.append(M)
        As.append(A)
        work = jnp.where(iota_l == A[:, None, :], neg, work)
    M5 = jnp.max(work, axis=1)

    V, Ac = Ms[0], As[0]
    cnt = jnp.zeros((_RB, _C), jnp.int32)
    iota_c = lax.broadcasted_iota(jnp.int32, (_RB, _C), 1)
    cols = []
    vlast = None
    for _ in range(_K):
        cstar = jnp.argmax(V, axis=1).astype(jnp.int32)
        astar = jnp.take_along_axis(Ac, cstar[:, None], axis=1)
        vlast = jnp.take_along_axis(V, cstar[:, None], axis=1)
        cols.append(astar * _C + cstar[:, None])
        onehot = iota_c == cstar[:, None]
        cnt = cnt + onehot.astype(jnp.int32)
        nxtV = jnp.where(cnt == 1, Ms[1],
                         jnp.where(cnt == 2, Ms[2],
                                   jnp.where(cnt == 3, Ms[3], neg)))
        nxtA = jnp.where(cnt == 1, As[1],
                         jnp.where(cnt == 2, As[2], As[3]))
        V = jnp.where(onehot, nxtV, V)
        Ac = jnp.where(onehot, nxtA, Ac)
    out_ref[...] = jnp.concatenate(cols, axis=1)

    overflow = jnp.any((cnt >= _DEPTH) & (M5 > vlast))

    @pl.when(overflow)
    def _fallback():
        raw2 = lax.dot_general(eb, ef, (((1,), (1,)), ((), ())),
                               preferred_element_type=jnp.float32)
        cos2 = raw2 / (nb[:, None] * nf[None, :])
        cos2 = jnp.where(iotc >= _N, -2.0, cos2)
        colsf = []
        for _ in range(_K):
            idx = jnp.argmax(cos2, axis=1).astype(jnp.int32)
            colsf.append(idx[:, None])
            cos2 = jnp.where(iotc == idx[:, None], neg, cos2)
        out_ref[...] = jnp.concatenate(colsf, axis=1)


def _topk_stage(emb):
    emb_pad = jnp.pad(emb, ((0, _NPAD - _N), (0, 0)))
    return pl.pallas_call(
        _topk_body,
        grid=(_NBLK,),
        in_specs=[pl.BlockSpec((_RB, _ED), lambda i: (i, 0)),
                  pl.BlockSpec((_NPAD, _ED), lambda i: (0, 0))],
        out_specs=pl.BlockSpec((_RB, _K), lambda i: (i, 0)),
        out_shape=jax.ShapeDtypeStruct((_N, _K), jnp.int32),
    )(emb, emb_pad)


def _feat_body(x_ref, emb_ref, wg_ref, atts_ref, z_ref, ai_ref, aj_ref):
    x = x_ref[0]
    e = emb_ref[...]
    z = jnp.dot(x, wg_ref[...], preferred_element_type=jnp.float32)
    atts = atts_ref[...]
    ai = jnp.sum(z * atts[0][None, :], axis=1) + jnp.sum(e * atts[1][None, :], axis=1)
    aj = jnp.sum(z * atts[2][None, :], axis=1) + jnp.sum(e * atts[3][None, :], axis=1)
    z_ref[0] = z
    ai_ref[0, 0] = ai
    aj_ref[0, 0] = aj


def _feat_stage(data, emb, W_gat, atts):
    B_, N_, Ft = data.shape
    return pl.pallas_call(
        _feat_body,
        grid=(B_,),
        in_specs=[pl.BlockSpec((1, N_, Ft), lambda b: (b, 0, 0)),
                  pl.BlockSpec((N_, _ED), lambda b: (0, 0)),
                  pl.BlockSpec((Ft, _ED), lambda b: (0, 0)),
                  pl.BlockSpec((4, _ED), lambda b: (0, 0))],
        out_specs=[pl.BlockSpec((1, N_, _ED), lambda b: (b, 0, 0)),
                   pl.BlockSpec((1, 1, N_), lambda b: (b, 0, 0)),
                   pl.BlockSpec((1, 1, N_), lambda b: (b, 0, 0))],
        out_shape=[jax.ShapeDtypeStruct((B_, N_, _ED), jnp.float32),
                   jax.ShapeDtypeStruct((B_, 1, N_), jnp.float32),
                   jax.ShapeDtypeStruct((B_, 1, N_), jnp.float32)],
    )(data, emb, W_gat, atts)


def _lane_bcast(vec, idx16):
    return lax.gather(
        vec, idx16[:, None],
        lax.GatherDimensionNumbers(offset_dims=(), collapsed_slice_dims=(0,),
                                   start_index_map=(0,)),
        (1,), mode=lax.GatherScatterMode.PROMISE_IN_BOUNDS)


def _gat_body(J_hbm, z_hbm, ai_hbm, aj_hbm, emb_hbm, epar_hbm, out_hbm,
              Jv, ajv, aiv, embv, zg, idxv, eparv, outv, sem):
    b = lax.axis_index("c")
    s = lax.axis_index("s")
    i0 = s * _PW
    pltpu.sync_copy(aj_hbm.at[b], ajv)
    pltpu.sync_copy(ai_hbm.at[b, pl.ds(i0, _PW)], aiv)
    pltpu.sync_copy(emb_hbm.at[pl.ds(i0, _PW)], embv)
    pltpu.sync_copy(epar_hbm, eparv)
    for k in range(_K):
        pltpu.sync_copy(J_hbm.at[k, pl.ds(i0, _PW)], Jv.at[k])
    iota16 = lax.iota(jnp.int32, 16)
    zoff = b * _NP

    def _start_gather(g_next, buf):
        for k in range(_K):
            jv = Jv[k, pl.ds(g_next * 16, 16)]
            idxv[buf, k // 8, pl.ds((k % 8) * 16, 16)] = jv + zoff
        for r in range(2):
            pltpu.async_copy(z_hbm.at[idxv.at[buf, r]], zg.at[buf, r], sem)

    def _wait_gather(buf):
        for r in range(2):
            pltpu.make_async_copy(
                z_hbm.at[idxv.at[buf, r]], zg.at[buf, r], sem).wait()

    _start_gather(0, 0)

    def _do_group(g, buf):
        gn = jnp.minimum(g + 1, _NG - 1)
        _start_gather(gn, 1 - buf)
        ai_v = aiv[pl.ds(g * 16, 16)]
        es = []
        for k in range(_K):
            jv = Jv[k, pl.ds(g * 16, 16)]
            ajg = plsc.load_gather(ajv, [jv])
            e = ai_v + ajg
            es.append(jnp.where(e > 0.0, e, 0.2 * e))
        m = es[0]
        for k in range(1, _K):
            m = jnp.maximum(m, es[k])
        exs = [jnp.exp(e - m) for e in es]
        ssum = exs[0]
        for k in range(1, _K):
            ssum = ssum + exs[k]
        rinv = 1.0 / ssum
        alphas = [ex * rinv for ex in exs]
        _wait_gather(buf)

        bg = [eparv[0, pl.ds(d * 16, 16)] for d in range(4)]
        gi = [eparv[1, pl.ds(d * 16, 16)] for d in range(4)]
        bt = [eparv[2, pl.ds(d * 16, 16)] for d in range(4)]
        wo = [eparv[3, pl.ds(d * 16, 16)] for d in range(4)]
        out_acc = plsc.load_gather(
            eparv, [jnp.full((16,), 4, jnp.int32),
                    jnp.zeros((16,), jnp.int32)])
        for n in range(16):
            nsel = jnp.full((16,), n, jnp.int32)
            accs = [jnp.zeros((16,), jnp.float32) for _ in range(4)]
            for k in range(_K):
                a = _lane_bcast(alphas[k], nsel)
                row = (k % 8) * 16 + n
                for d in range(4):
                    zr = zg[buf, k // 8, row, pl.ds(d * 16, 16)]
                    accs[d] = accs[d] + a * zr
            erow = g * 16 + n
            t = None
            for d in range(4):
                emb_d = embv[erow, pl.ds(d * 16, 16)]
                gat_d = accs[d] + bg[d]
                rst_d = jnp.maximum((gat_d * emb_d) * gi[d] + bt[d], 0.0)
                pd = rst_d * wo[d]
                t = pd if t is None else t + pd
            tot = _lane_bcast(plsc.cumsum(t), jnp.full((16,), 15, jnp.int32))
            out_acc = jnp.where(iota16 == n, out_acc + tot, out_acc)
        outv[pl.ds(g * 16, 16)] = out_acc

    def step(t, carry):
        _do_group(2 * t, 0)
        _do_group(2 * t + 1, 1)
        return carry

    lax.fori_loop(0, _NG // 2, step, 0)
    _wait_gather(0)
    pltpu.sync_copy(outv, out_hbm.at[b, pl.ds(i0, _PW)])


def _gat_call(*args):
    fn = functools.partial(
        pl.kernel,
        out_type=jax.ShapeDtypeStruct((2, _NP), jnp.float32),
        mesh=plsc.VectorSubcoreMesh(core_axis_name="c", subcore_axis_name="s"),
        compiler_params=pltpu.CompilerParams(needs_layout_passes=False,
                                             use_tc_tiling_on_sc=False),
        scratch_types=[
            pltpu.VMEM((_K, _PW), jnp.int32),
            pltpu.VMEM((_NP,), jnp.float32),
            pltpu.VMEM((_PW,), jnp.float32),
            pltpu.VMEM((_PW, _ED), jnp.float32),
            pltpu.VMEM((2, 2, 128, _ED), jnp.float32),
            pltpu.VMEM((2, 2, 128), jnp.int32),
            pltpu.VMEM((8, _ED), jnp.float32),
            pltpu.VMEM((_PW,), jnp.float32),
            pltpu.SemaphoreType.DMA,
        ],
    )(_gat_body)
    return fn(*args)


def kernel(data, emb, W_gat, att_i, att_j, b_gat, gamma, beta, W_out, b_out):
    B_, N_, Ft = data.shape
    topk = _topk_stage(emb)
    atts = jnp.stack([att_i[:_ED], att_i[_ED:], att_j[:_ED], att_j[_ED:]])
    z, ai3, aj3 = _feat_stage(data, emb, W_gat, atts)
    ai2 = ai3.reshape(B_, N_)
    aj2 = aj3.reshape(B_, N_)
    pad = _NP - N_
    J = jnp.pad(topk.reshape(_K, N_), ((0, 0), (0, pad)))
    z_p = jnp.pad(z, ((0, 0), (0, pad), (0, 0))).reshape(B_ * _NP, _ED)
    ai_p = jnp.pad(ai2, ((0, 0), (0, pad)))
    aj_p = jnp.pad(aj2, ((0, 0), (0, pad)))
    emb_p = jnp.pad(emb, ((0, pad), (0, 0)))
    ginv = gamma * np.float32(1.0 / np.sqrt(1.0 + 1e-5))
    epar = jnp.stack([b_gat, ginv, beta, W_out[:, 0],
                      jnp.full((_ED,), b_out[0], jnp.float32),
                      jnp.zeros((_ED,), jnp.float32),
                      jnp.zeros((_ED,), jnp.float32),
                      jnp.zeros((_ED,), jnp.float32)])
    out_p = _gat_call(J, z_p, ai_p, aj_p, emb_p, epar)
    return out_p[:, :N_]
